# Initial kernel scaffold; baseline (speedup 1.0000x reference)
import jax
import jax.numpy as jnp
from jax.experimental import pallas as pl


def kernel(x, edge_index, W0, a_src0, a_dst0, bias0, bn_g0, bn_b0, W1, a_src1, a_dst1, bias1, bn_g1, bn_b1, W2, a_src2, a_dst2, bias2, bn_g2, bn_b2, lin1_W, lin1_b, lin_W, lin_b):
    raise NotImplementedError("write your pallas kernel here")


# speedup vs baseline: 11.6944x; 11.6944x over previous
import functools

import jax
import jax.numpy as jnp
from jax import lax
from jax.experimental import pallas as pl
from jax.experimental.pallas import tpu as pltpu
from jax.experimental.pallas import tpu_sc as plsc

N = 10000
D = 128
E = 320000
E2 = E + N
NP = 10240
NC = 2
NS = 16
NW = NC * NS
L = 16
K = 128
BPT = 2 * (-(-E2 // (NW * K * 2)))
EP = NW * BPT * K
EP_PAD = EP + 2 * K
TR = NP // D
RPT = NP // NS
CH = 80


def _sc_edge_body(hp_hbm, asv_hbm, adv_hbm, src_hbm, dst_hbm,
                  feat_hbm, srep_hbm,
                  as_v, ad_v, si_v, di_v, rows_v, e_v, sp_v,
                  acc_sh, sem_row, sem_si, sem_di):
    cid = lax.axis_index("c")
    sid = lax.axis_index("s")
    wid = sid * NC + cid
    base_row = wid * BPT

    pltpu.sync_copy(asv_hbm, as_v)
    pltpu.sync_copy(adv_hbm, ad_v)

    zero16 = jnp.zeros((L,), jnp.float32)
    lane = lax.iota(jnp.int32, L)

    def _zsp(i, c):
        for g in range(D // L):
            sp_v[i, pl.ds(g * L, L)] = zero16
        return c
    lax.fori_loop(0, TR, _zsp, 0)

    nch = jnp.where(sid < NS - 1, RPT // CH, (N - (NS - 1) * RPT) // CH)

    def _zacc(i, c):
        pltpu.sync_copy(sp_v, acc_sh.at[pl.ds(sid * RPT + i * CH, CH)])
        return c
    lax.fori_loop(0, nch, _zacc, 0)

    plsc.subcore_barrier()

    for p in range(2):
        pltpu.async_copy(src_hbm.at[pl.ds((base_row + p) * K, K)],
                         si_v.at[p], sem_si)
        pltpu.async_copy(dst_hbm.at[pl.ds((base_row + p) * K, K)],
                         di_v.at[p], sem_di)

    def _pair(t, carry):
        for p in range(2):
            j = 2 * t + p
            pltpu.make_async_copy(src_hbm.at[pl.ds(0, K)], si_v.at[p],
                                  sem_si).wait()
            pltpu.make_async_copy(dst_hbm.at[pl.ds(0, K)], di_v.at[p],
                                  sem_di).wait()
            cp = pltpu.async_copy(hp_hbm.at[si_v.at[p]], rows_v, sem_row)

            for g in range(K // L):
                s16 = si_v[p, pl.ds(g * L, L)]
                d16 = di_v[p, pl.ds(g * L, L)]
                a1 = plsc.load_gather(as_v, [s16 >> 7, s16 & 127])
                a2 = plsc.load_gather(ad_v, [d16 >> 7, d16 & 127])
                al = a1 + a2
                al = jnp.where(al > 0.0, al, al * jnp.float32(0.2))
                ev = jnp.exp(al)
                gidx = (base_row + j) * K + g * L + lane
                ev = jnp.where(gidx < E2, ev, jnp.float32(0.0))
                e_v[pl.ds(g * L, L)] = ev
                for i in range(L):
                    plsc.addupdate_scatter(sp_v, [d16 >> 7, d16 & 127],
                                           ev, mask=lane == i)

            cp.wait()

            def _scale(k, c2):
                espl = plsc.load_gather(e_v, [jnp.full((L,), k, jnp.int32)])
                for g in range(D // L):
                    sl = pl.ds(g * L, L)
                    rows_v[k, sl] = rows_v[k, sl] * espl
                return c2
            lax.fori_loop(0, K, _scale, 0)

            pltpu.sync_copy(rows_v, acc_sh.at[di_v.at[p]], add=True)

            pltpu.async_copy(src_hbm.at[pl.ds((base_row + j + 2) * K, K)],
                             si_v.at[p], sem_si)
            pltpu.async_copy(dst_hbm.at[pl.ds((base_row + j + 2) * K, K)],
                             di_v.at[p], sem_di)
        return carry

    lax.fori_loop(0, BPT // 2, _pair, 0)

    for p in range(2):
        pltpu.make_async_copy(src_hbm.at[pl.ds(0, K)], si_v.at[p],
                              sem_si).wait()
        pltpu.make_async_copy(dst_hbm.at[pl.ds(0, K)], di_v.at[p],
                              sem_di).wait()

    plsc.subcore_barrier()

    def _fout(i, c):
        sl = pl.ds(sid * RPT + i * CH, CH)
        pltpu.sync_copy(acc_sh.at[sl], feat_hbm.at[cid, sl])
        return c
    lax.fori_loop(0, nch, _fout, 0)

    plsc.subcore_barrier()

    pltpu.sync_copy(sp_v, acc_sh.at[pl.ds(sid * TR, TR)])
    plsc.subcore_barrier()

    def _zss(i, c):
        for g in range(D // L):
            ad_v[i, pl.ds(g * L, L)] = zero16
        return c
    lax.fori_loop(0, TR, _zss, 0)

    def _sacc(tt, c):
        pltpu.sync_copy(acc_sh.at[pl.ds(tt * TR, TR)], as_v)

        def _srow(i, c2):
            for g in range(D // L):
                sl = pl.ds(g * L, L)
                ad_v[i, sl] = ad_v[i, sl] + as_v[i, sl]
            return c2
        lax.fori_loop(0, TR, _srow, 0)
        return c
    lax.fori_loop(0, NS, _sacc, 0)

    def _rchunk(ci, c):
        n0 = sid * RPT + ci * CH

        def _rrow(rr, c2):
            nn = n0 + rr
            spl = plsc.load_gather(
                ad_v, [jnp.full((L,), nn >> 7, jnp.int32),
                       jnp.full((L,), nn & 127, jnp.int32)])
            for g in range(D // L):
                sp_v[rr, pl.ds(g * L, L)] = spl
            return c2
        lax.fori_loop(0, CH, _rrow, 0)
        pltpu.sync_copy(sp_v, srep_hbm.at[cid, pl.ds(n0, CH)])
        return c
    lax.fori_loop(0, nch, _rchunk, 0)


@functools.cache
def _sc_edge_kernel():
    return functools.partial(
        pl.kernel,
        out_type=[
            jax.ShapeDtypeStruct((NC, NP, D), jnp.float32),
            jax.ShapeDtypeStruct((NC, NP, D), jnp.float32),
        ],
        mesh=plsc.VectorSubcoreMesh(core_axis_name="c", subcore_axis_name="s",
                                    num_cores=NC, num_subcores=NS),
        compiler_params=pltpu.CompilerParams(needs_layout_passes=False),
        scratch_types=[
            pltpu.VMEM((TR, D), jnp.float32),
            pltpu.VMEM((TR, D), jnp.float32),
            pltpu.VMEM((2, K), jnp.int32),
            pltpu.VMEM((2, K), jnp.int32),
            pltpu.VMEM((K, D), jnp.float32),
            pltpu.VMEM((K,), jnp.float32),
            pltpu.VMEM((TR, D), jnp.float32),
            pltpu.VMEM_SHARED((N, D), jnp.float32),
            pltpu.SemaphoreType.DMA,
            pltpu.SemaphoreType.DMA,
            pltpu.SemaphoreType.DMA,
        ],
    )(_sc_edge_body)


def _sc_edge(hp, asv, adv, src1, dst1):
    return _sc_edge_kernel()(hp, asv, adv, src1, dst1)


def _tc_first_body(x_ref, w_ref, asw_ref, adw_ref, hp_ref, asv_ref, adv_ref):
    hp = jnp.dot(x_ref[...], w_ref[...], preferred_element_type=jnp.float32)
    hp_ref[...] = hp
    asv_ref[...] = jnp.sum(hp * asw_ref[0], axis=1, keepdims=True)
    adv_ref[...] = jnp.sum(hp * adw_ref[0], axis=1, keepdims=True)


def _tc_first(xp, W, a_s, a_d):
    return pl.pallas_call(
        _tc_first_body,
        out_shape=[
            jax.ShapeDtypeStruct((NP, D), jnp.float32),
            jax.ShapeDtypeStruct((NP, 1), jnp.float32),
            jax.ShapeDtypeStruct((NP, 1), jnp.float32),
        ],
    )(xp, W, a_s.reshape(1, D), a_d.reshape(1, D))


def _norm_bn_relu(pf_ref, ps_ref, bias_ref, g_ref, b_ref):
    raw = pf_ref[0]
    s = ps_ref[0]
    for c in range(1, NC):
        raw = raw + pf_ref[c]
        s = s + ps_ref[c]
    Z = raw[:N] / (s[:N] + jnp.float32(1e-16)) + bias_ref[0]
    mu = jnp.mean(Z, axis=0)
    var = jnp.mean((Z - mu) ** 2, axis=0)
    h = (Z - mu) / jnp.sqrt(var + jnp.float32(1e-5)) * g_ref[0] + b_ref[0]
    return jnp.maximum(h, jnp.float32(0.0))


def _tc_mid_body(pf_ref, ps_ref, bias_ref, g_ref, b_ref, w_ref, asw_ref,
                 adw_ref, hp_ref, asv_ref, adv_ref):
    h = _norm_bn_relu(pf_ref, ps_ref, bias_ref, g_ref, b_ref)
    hp10 = jnp.dot(h, w_ref[...], preferred_element_type=jnp.float32)
    hp = jnp.concatenate(
        [hp10, jnp.zeros((NP - N, D), jnp.float32)], axis=0)
    hp_ref[...] = hp
    asv_ref[...] = jnp.sum(hp * asw_ref[0], axis=1, keepdims=True)
    adv_ref[...] = jnp.sum(hp * adw_ref[0], axis=1, keepdims=True)


def _tc_mid(pf, ps, bias, g, b, W, a_s, a_d):
    return pl.pallas_call(
        _tc_mid_body,
        out_shape=[
            jax.ShapeDtypeStruct((NP, D), jnp.float32),
            jax.ShapeDtypeStruct((NP, 1), jnp.float32),
            jax.ShapeDtypeStruct((NP, 1), jnp.float32),
        ],
    )(pf, ps, bias.reshape(1, D), g.reshape(1, D), b.reshape(1, D), W,
      a_s.reshape(1, D), a_d.reshape(1, D))


def _tc_last_body(pf_ref, ps_ref, bias_ref, g_ref, b_ref, w1_ref, b1_ref,
                  w2_ref, b2_ref, out_ref):
    h = _norm_bn_relu(pf_ref, ps_ref, bias_ref, g_ref, b_ref)
    h = jnp.maximum(jnp.dot(h, w1_ref[...],
                            preferred_element_type=jnp.float32)
                    + b1_ref[0], jnp.float32(0.0))
    h = jnp.maximum(jnp.dot(h, w1_ref[...],
                            preferred_element_type=jnp.float32)
                    + b1_ref[0], jnp.float32(0.0))
    out_ref[...] = jnp.dot(h, w2_ref[...],
                           preferred_element_type=jnp.float32) + b2_ref[0]


def _tc_last(pf, ps, bias, g, b, lin1_W, lin1_b, lin_W, lin_b):
    return pl.pallas_call(
        _tc_last_body,
        out_shape=jax.ShapeDtypeStruct((N, D), jnp.float32),
    )(pf, ps, bias.reshape(1, D), g.reshape(1, D), b.reshape(1, D),
      lin1_W, lin1_b.reshape(1, D), lin_W, lin_b.reshape(1, D))


def kernel(x, edge_index, W0, a_src0, a_dst0, bias0, bn_g0, bn_b0,
           W1, a_src1, a_dst1, bias1, bn_g1, bn_b1,
           W2, a_src2, a_dst2, bias2, bn_g2, bn_b2,
           lin1_W, lin1_b, lin_W, lin_b):
    idt = edge_index.dtype
    loop = jnp.arange(N, dtype=idt)
    pad = jnp.zeros((EP_PAD - E2,), dtype=idt)
    src1 = jnp.concatenate([edge_index[0], loop, pad])
    dst1 = jnp.concatenate([edge_index[1], loop, pad])
    xp = jnp.concatenate([x, jnp.zeros((NP - N, D), x.dtype)], axis=0)

    def table(v):
        return v.reshape(TR, D)

    hp, asv, adv = _tc_first(xp, W0, a_src0, a_dst0)
    pf, ps = _sc_edge(hp, table(asv), table(adv), src1, dst1)
    hp, asv, adv = _tc_mid(pf, ps, bias0, bn_g0, bn_b0, W1, a_src1, a_dst1)
    pf, ps = _sc_edge(hp, table(asv), table(adv), src1, dst1)
    hp, asv, adv = _tc_mid(pf, ps, bias1, bn_g1, bn_b1, W2, a_src2, a_dst2)
    pf, ps = _sc_edge(hp, table(asv), table(adv), src1, dst1)
    return _tc_last(pf, ps, bias2, bn_g2, bn_b2, lin1_W, lin1_b, lin_W,
                    lin_b)
